# Initial kernel scaffold; baseline (speedup 1.0000x reference)
import jax
import jax.numpy as jnp
from jax.experimental import pallas as pl


def kernel(idx, logits):
    raise NotImplementedError("write your pallas kernel here")


# speedup vs baseline: 1.0057x; 1.0057x over previous
import functools

import jax
import jax.numpy as jnp
from jax import lax
from jax.experimental import pallas as pl
from jax.experimental.pallas import tpu as pltpu
from jax.experimental.pallas import tpu_sc as plsc

_VOCAB = 1000
_NTOK = 1024 * 50
_NC, _NS = 2, 16
_NW = _NC * _NS
_PER_W = _NTOK // _NW
_CHUNK = 64
_NCHUNK = _PER_W // _CHUNK


def _make_gather():
    mesh = plsc.VectorSubcoreMesh(core_axis_name="c", subcore_axis_name="s")

    @functools.partial(
        pl.kernel,
        mesh=mesh,
        compiler_params=pltpu.CompilerParams(use_tc_tiling_on_sc=False),
        out_type=jax.ShapeDtypeStruct((_NTOK, _VOCAB), jnp.float32),
        scratch_types=[
            pltpu.VMEM((_CHUNK,), jnp.int32),
            pltpu.VMEM((_CHUNK, _VOCAB), jnp.float32),
            pltpu.SemaphoreType.DMA,
        ],
    )
    def gather_kernel(idx_hbm, table_hbm, out_hbm, idx_v, rows_v, sem):
        wid = lax.axis_index("s") * _NC + lax.axis_index("c")
        base = wid * _PER_W

        def body(j, carry):
            off = base + j * _CHUNK
            pltpu.sync_copy(idx_hbm.at[pl.ds(off, _CHUNK)], idx_v)
            pltpu.async_copy(table_hbm.at[idx_v], rows_v, sem).wait()
            pltpu.sync_copy(rows_v, out_hbm.at[pl.ds(off, _CHUNK)])
            return carry

        lax.fori_loop(0, _NCHUNK, body, 0)

    return gather_kernel


_gather = _make_gather()


@jax.jit
def kernel(idx, logits):
    flat = idx.reshape(_NTOK)
    out = _gather(flat, logits)
    return out.reshape(idx.shape[0], idx.shape[1], _VOCAB)


# speedup vs baseline: 1.0089x; 1.0032x over previous
import functools

import jax
import jax.numpy as jnp
from jax import lax
from jax.experimental import pallas as pl
from jax.experimental.pallas import tpu as pltpu
from jax.experimental.pallas import tpu_sc as plsc

_VOCAB = 1000
_NTOK = 1024 * 50
_NC, _NS = 2, 16
_NW = _NC * _NS
_PER_W = _NTOK // _NW
_CHUNK = 64
_NCHUNK = _PER_W // _CHUNK


def _make_gather():
    mesh = plsc.VectorSubcoreMesh(core_axis_name="c", subcore_axis_name="s")

    @functools.partial(
        pl.kernel,
        mesh=mesh,
        compiler_params=pltpu.CompilerParams(use_tc_tiling_on_sc=False),
        out_type=jax.ShapeDtypeStruct((_NTOK, _VOCAB), jnp.float32),
        scratch_types=[
            pltpu.VMEM((_PER_W,), jnp.int32),
            pltpu.VMEM((_CHUNK, _VOCAB), jnp.float32),
            pltpu.VMEM((_CHUNK, _VOCAB), jnp.float32),
            pltpu.SemaphoreType.DMA,
            pltpu.SemaphoreType.DMA,
            pltpu.SemaphoreType.DMA,
            pltpu.SemaphoreType.DMA,
        ],
    )
    def gather_kernel(idx_hbm, table_hbm, out_hbm, idx_v, bufa, bufb,
                      ga, gb, wa, wb):
        wid = lax.axis_index("s") * _NC + lax.axis_index("c")
        base = wid * _PER_W
        pltpu.sync_copy(idx_hbm.at[pl.ds(base, _PER_W)], idx_v)

        def idx_slice(j):
            return idx_v.at[pl.ds(j * _CHUNK, _CHUNK)]

        def out_slice(j):
            return out_hbm.at[pl.ds(base + j * _CHUNK, _CHUNK)]

        def start_gather(j, buf, sem):
            return pltpu.async_copy(table_hbm.at[idx_slice(j)], buf, sem)

        def start_write(j, buf, sem):
            return pltpu.async_copy(buf, out_slice(j), sem)

        def wait_write(buf, sem):
            pltpu.make_async_copy(buf, out_slice(0), sem).wait()

        g0 = start_gather(0, bufa, ga)
        g1 = start_gather(1, bufb, gb)
        g0.wait()
        start_write(0, bufa, wa)
        g1.wait()
        start_write(1, bufb, wb)

        def body(t, carry):
            j = 2 * t
            wait_write(bufa, wa)
            gac = start_gather(j, bufa, ga)
            wait_write(bufb, wb)
            gbc = start_gather(j + 1, bufb, gb)
            gac.wait()
            start_write(j, bufa, wa)
            gbc.wait()
            start_write(j + 1, bufb, wb)
            return carry

        lax.fori_loop(1, _NCHUNK // 2, body, 0)

        wait_write(bufa, wa)
        gl = start_gather(_NCHUNK - 1, bufa, ga)
        wait_write(bufb, wb)
        gl.wait()
        start_write(_NCHUNK - 1, bufa, wa)
        wait_write(bufa, wa)

    return gather_kernel


_gather = _make_gather()


@jax.jit
def kernel(idx, logits):
    flat = idx.reshape(_NTOK)
    out = _gather(flat, logits)
    return out.reshape(idx.shape[0], idx.shape[1], _VOCAB)


# speedup vs baseline: 1.3804x; 1.3683x over previous
import functools

import jax
import jax.numpy as jnp
from jax import lax
from jax.experimental import pallas as pl
from jax.experimental.pallas import tpu as pltpu
from jax.experimental.pallas import tpu_sc as plsc

_VOCAB = 1000
_B, _L = 1024, 50
_NC, _NS = 2, 16
_NW = _NC * _NS
_BPW = _B // _NW
_NBLK = _VOCAB // 128
_TAIL = _VOCAB - 128 * _NBLK
_LP = 56


def _make_gather():
    mesh = plsc.VectorSubcoreMesh(core_axis_name="c", subcore_axis_name="s")

    @functools.partial(
        pl.kernel,
        mesh=mesh,
        out_type=jax.ShapeDtypeStruct((_B, _L, _VOCAB), jnp.float32),
        scratch_types=[
            pltpu.VMEM((8 * _LP,), jnp.int32),
            pltpu.VMEM((_L, 128 * _NBLK), jnp.float32),
            pltpu.VMEM((_L, 128), jnp.float32),
            pltpu.SemaphoreType.DMA,
            pltpu.SemaphoreType.DMA,
            pltpu.SemaphoreType.DMA,
        ],
    )
    def gather_kernel(idxm_hbm, table_hbm, out_hbm, idx_v, buf, tail,
                      g, gt, w):
        wid = lax.axis_index("s") * _NC + lax.axis_index("c")
        b0 = wid * _BPW

        def body(k, carry):
            b = b0 + k
            pltpu.sync_copy(idxm_hbm.at[pl.ds(b * 8 * _LP, 8 * _LP)], idx_v)
            copies = []
            for c in range(_NBLK):
                copies.append(pltpu.async_copy(
                    table_hbm.at[idx_v.at[pl.ds(c * _LP, _L)]],
                    buf.at[:, pl.ds(c * 128, 128)], g))
            tc = pltpu.async_copy(
                table_hbm.at[idx_v.at[pl.ds(_NBLK * _LP, _L)]], tail, gt)
            tc.wait()
            def tail_out(r, rcarry):
                pltpu.sync_copy(
                    tail.at[r, pl.ds(0, _TAIL)],
                    out_hbm.at[b, r, pl.ds(128 * _NBLK, _TAIL)])
                return rcarry
            lax.fori_loop(0, _L, tail_out, 0)
            for cp in copies:
                cp.wait()
            pltpu.async_copy(
                buf, out_hbm.at[b, :, pl.ds(0, 128 * _NBLK)], w).wait()
            return carry

        lax.fori_loop(0, _BPW, body, 0)

    return gather_kernel


_gather = _make_gather()


@jax.jit
def kernel(idx, logits):
    table_p = jnp.pad(logits, ((0, 0), (0, 24))).reshape(8 * _VOCAB, 128)
    idxm = (idx * 8)[:, None, :] + jnp.arange(8, dtype=idx.dtype)[None, :, None]
    idxm = jnp.pad(idxm, ((0, 0), (0, 0), (0, _LP - _L))).reshape(-1)
    return _gather(idxm, table_p)
